# baseline (speedup 1.0000x reference)
import functools

import jax
import jax.numpy as jnp
from jax import lax
from jax.experimental import pallas as pl
from jax.experimental.pallas import tpu as pltpu
from jax.experimental.pallas import tpu_sc as plsc

_NUM_OMEGA = 1000
_NUM_SIGMA = 32
_N = 65536
_TOL = 1e-7

_NC = 2
_NS = 16
_L = 16
_NW = _NC * _NS
_CHUNK = _N // _NW
_NVEC = _CHUNK // _L

_P_COEF = (1.0, -0.1666666716337204, 0.008333330973982811,
           -0.00019840861205011606, 2.752528644123231e-06,
           -2.3889498379503493e-08)
_C_COEF = (1.0, -0.5, 0.0416666679084301, -0.0013888884568586946,
           2.4801040126476437e-05, -2.75246833325582e-07,
           1.990768216941774e-09)

_mesh = plsc.VectorSubcoreMesh(core_axis_name="c", subcore_axis_name="s")


def _horner(coef, t):
    acc = jnp.full((_L,), coef[-1], jnp.float32)
    for c in coef[-2::-1]:
        acc = acc * t + c
    return acc


def _rsqrt(n2):
    i = plsc.bitcast(n2, jnp.int32)
    g = plsc.bitcast(jnp.int32(0x5F3759DF) - (i >> 1), jnp.float32)
    for _ in range(2):
        g = g * (1.5 - 0.5 * n2 * g * g)
    return g


@functools.partial(
    pl.kernel,
    out_type=jax.ShapeDtypeStruct((9, _N), jnp.float32),
    mesh=_mesh,
    compiler_params=pltpu.CompilerParams(needs_layout_passes=False),
    scratch_types=[
        pltpu.VMEM((_NUM_OMEGA * 33,), jnp.float32),
        pltpu.VMEM((_NUM_OMEGA * 33,), jnp.float32),
        pltpu.VMEM((_CHUNK,), jnp.int32),
        pltpu.VMEM((_CHUNK,), jnp.float32),
        pltpu.VMEM((3, _CHUNK), jnp.float32),
        pltpu.VMEM((9, _CHUNK), jnp.float32),
        pltpu.SemaphoreType.DMA,
    ],
)
def _sc_fused(sig_hbm, u_hbm, ax_hbm, cdf_hbm, og_hbm, out_hbm,
              cdf_v, og_v, sig_v, u_v, ax_v, out_v, sem):
    wid = lax.axis_index("s") * _NC + lax.axis_index("c")
    base = wid * _CHUNK
    copies = [
        pltpu.async_copy(cdf_hbm, cdf_v, sem),
        pltpu.async_copy(og_hbm, og_v, sem),
        pltpu.async_copy(sig_hbm.at[pl.ds(base, _CHUNK)], sig_v, sem),
        pltpu.async_copy(u_hbm.at[pl.ds(base, _CHUNK)], u_v, sem),
        pltpu.async_copy(ax_hbm.at[:, pl.ds(base, _CHUNK)], ax_v, sem),
    ]
    for c in copies:
        c.wait()

    def body(i):
        off = pl.multiple_of(i * _L, _L)
        s = sig_v[pl.ds(off, _L)]
        uu = u_v[pl.ds(off, _L)]
        pos = jnp.zeros((_L,), jnp.int32)
        for step in (512, 256, 128, 64, 32):
            cand = pos + step
            val = plsc.load_gather(cdf_v, [cand * 33 + s])
            pos = jnp.where(val < uu, cand, pos)
        for step in (16, 8, 4, 2, 1):
            cand = pos + step
            candc = jnp.minimum(cand, _NUM_OMEGA - 1)
            val = plsc.load_gather(cdf_v, [candc * 33 + s])
            take = jnp.logical_and(cand <= _NUM_OMEGA - 1, val < uu)
            pos = jnp.where(take, cand, pos)
        idx = pos + 1
        c_lo = plsc.load_gather(cdf_v, [pos * 33 + s])
        c_hi = plsc.load_gather(cdf_v, [idx * 33 + s])
        o_lo = plsc.load_gather(og_v, [pos * 33])
        o_hi = plsc.load_gather(og_v, [idx * 33])
        denom = jnp.maximum(c_hi - c_lo, 1e-10)
        om = o_lo + (uu - c_lo) * (o_hi - o_lo) / denom

        x = ax_v[0, pl.ds(off, _L)]
        y = ax_v[1, pl.ds(off, _L)]
        z = ax_v[2, pl.ds(off, _L)]
        n2 = x * x + y * y + z * z
        g = _rsqrt(jnp.maximum(n2, 1e-30))
        nrm = n2 * g
        scale = om / (nrm + _TOL)
        rx = x * scale
        ry = y * scale
        rz = z * scale
        t = (rx * rx + ry * ry + rz * rz) * 0.25
        pv = _horner(_P_COEF, t)
        cv = _horner(_C_COEF, t)
        sin_c = pv * cv
        cos_c = 0.5 * (pv * pv)
        xx = rx * rx
        yy = ry * ry
        zz = rz * rz
        xy = rx * ry
        xz = rx * rz
        yz = ry * rz
        vals = (1.0 - cos_c * (yy + zz), cos_c * xy - sin_c * rz,
                cos_c * xz + sin_c * ry, cos_c * xy + sin_c * rz,
                1.0 - cos_c * (xx + zz), cos_c * yz - sin_c * rx,
                cos_c * xz - sin_c * ry, cos_c * yz + sin_c * rx,
                1.0 - cos_c * (xx + yy))
        for k, v in enumerate(vals):
            out_v[k, pl.ds(off, _L)] = v

    plsc.parallel_loop(0, _NVEC, unroll=8)(body)
    pltpu.sync_copy(out_v, out_hbm.at[:, pl.ds(base, _CHUNK)])


def kernel(sigma_indices, u, axis, cdf, omega_grid):
    out9 = _sc_fused(sigma_indices.astype(jnp.int32),
                     u.astype(jnp.float32),
                     axis.astype(jnp.float32).T,
                     jnp.pad(cdf.astype(jnp.float32).T,
                             ((0, 0), (0, 33 - _NUM_SIGMA))).reshape(-1),
                     jnp.pad(omega_grid.astype(jnp.float32)[:, None],
                             ((0, 0), (0, 32))).reshape(-1))
    return out9.T.reshape(_N, 3, 3)


# speedup vs baseline: 1.1071x; 1.1071x over previous
import functools

import jax
import jax.numpy as jnp
from jax import lax
from jax.experimental import pallas as pl
from jax.experimental.pallas import tpu as pltpu
from jax.experimental.pallas import tpu_sc as plsc

_NUM_OMEGA = 1000
_NUM_SIGMA = 32
_N = 65536
_TOL = 1e-7

_NC = 2
_NS = 16
_L = 16
_NW = _NC * _NS
_CHUNK = _N // _NW
_NVEC = _CHUNK // _L

_P_COEF = (1.0, -0.1666666716337204, 0.008333330973982811,
           -0.00019840861205011606, 2.752528644123231e-06,
           -2.3889498379503493e-08)
_C_COEF = (1.0, -0.5, 0.0416666679084301, -0.0013888884568586946,
           2.4801040126476437e-05, -2.75246833325582e-07,
           1.990768216941774e-09)

_mesh = plsc.VectorSubcoreMesh(core_axis_name="c", subcore_axis_name="s")


def _horner(coef, t):
    acc = jnp.full((_L,), coef[-1], jnp.float32)
    for c in coef[-2::-1]:
        acc = acc * t + c
    return acc


def _rsqrt(n2):
    i = plsc.bitcast(n2, jnp.int32)
    g = plsc.bitcast(jnp.int32(0x5F3759DF) - (i >> 1), jnp.float32)
    for _ in range(2):
        g = g * (1.5 - 0.5 * n2 * g * g)
    return g


@functools.partial(
    pl.kernel,
    out_type=jax.ShapeDtypeStruct((9, _N), jnp.float32),
    mesh=_mesh,
    compiler_params=pltpu.CompilerParams(needs_layout_passes=False),
    scratch_types=[
        pltpu.VMEM((_NUM_OMEGA * 33,), jnp.float32),
        pltpu.VMEM((_NUM_OMEGA,), jnp.float32),
        pltpu.VMEM((_CHUNK,), jnp.int32),
        pltpu.VMEM((_CHUNK,), jnp.float32),
        pltpu.VMEM((3, _CHUNK), jnp.float32),
        pltpu.VMEM((9, _CHUNK), jnp.float32),
        pltpu.SemaphoreType.DMA,
    ],
)
def _sc_fused(sig_hbm, u_hbm, ax_hbm, cdf_hbm, og_hbm, out_hbm,
              cdf_v, og_v, sig_v, u_v, ax_v, out_v, sem):
    wid = lax.axis_index("s") * _NC + lax.axis_index("c")
    base = wid * _CHUNK
    copies = [
        pltpu.async_copy(cdf_hbm, cdf_v, sem),
        pltpu.async_copy(og_hbm, og_v, sem),
        pltpu.async_copy(sig_hbm.at[pl.ds(base, _CHUNK)], sig_v, sem),
        pltpu.async_copy(u_hbm.at[pl.ds(base, _CHUNK)], u_v, sem),
        pltpu.async_copy(ax_hbm.at[:, pl.ds(base, _CHUNK)], ax_v, sem),
    ]
    for c in copies:
        c.wait()

    def body(i):
        off = pl.multiple_of(i * _L, _L)
        s = sig_v[pl.ds(off, _L)]
        uu = u_v[pl.ds(off, _L)]
        pos = jnp.zeros((_L,), jnp.int32)
        for step in (512, 256, 128, 64, 32):
            cand = pos + step
            val = plsc.load_gather(cdf_v, [cand * 33 + s])
            pos = jnp.where(val < uu, cand, pos)
        for step in (16, 8, 4, 2, 1):
            cand = pos + step
            candc = jnp.minimum(cand, _NUM_OMEGA - 1)
            val = plsc.load_gather(cdf_v, [candc * 33 + s])
            take = jnp.logical_and(cand <= _NUM_OMEGA - 1, val < uu)
            pos = jnp.where(take, cand, pos)
        idx = pos + 1
        c_lo = plsc.load_gather(cdf_v, [pos * 33 + s])
        c_hi = plsc.load_gather(cdf_v, [idx * 33 + s])
        o_lo = plsc.load_gather(og_v, [pos])
        o_hi = plsc.load_gather(og_v, [idx])
        denom = jnp.maximum(c_hi - c_lo, 1e-10)
        om = o_lo + (uu - c_lo) * (o_hi - o_lo) / denom

        x = ax_v[0, pl.ds(off, _L)]
        y = ax_v[1, pl.ds(off, _L)]
        z = ax_v[2, pl.ds(off, _L)]
        n2 = x * x + y * y + z * z
        g = _rsqrt(jnp.maximum(n2, 1e-30))
        nrm = n2 * g
        scale = om / (nrm + _TOL)
        rx = x * scale
        ry = y * scale
        rz = z * scale
        t = (rx * rx + ry * ry + rz * rz) * 0.25
        pv = _horner(_P_COEF, t)
        cv = _horner(_C_COEF, t)
        sin_c = pv * cv
        cos_c = 0.5 * (pv * pv)
        xx = rx * rx
        yy = ry * ry
        zz = rz * rz
        xy = rx * ry
        xz = rx * rz
        yz = ry * rz
        vals = (1.0 - cos_c * (yy + zz), cos_c * xy - sin_c * rz,
                cos_c * xz + sin_c * ry, cos_c * xy + sin_c * rz,
                1.0 - cos_c * (xx + zz), cos_c * yz - sin_c * rx,
                cos_c * xz - sin_c * ry, cos_c * yz + sin_c * rx,
                1.0 - cos_c * (xx + yy))
        for k, v in enumerate(vals):
            out_v[k, pl.ds(off, _L)] = v

    plsc.parallel_loop(0, _NVEC, unroll=8)(body)
    pltpu.sync_copy(out_v, out_hbm.at[:, pl.ds(base, _CHUNK)])


def kernel(sigma_indices, u, axis, cdf, omega_grid):
    out9 = _sc_fused(sigma_indices.astype(jnp.int32),
                     u.astype(jnp.float32),
                     axis.astype(jnp.float32).T,
                     jnp.pad(cdf.astype(jnp.float32).T,
                             ((0, 0), (0, 33 - _NUM_SIGMA))).reshape(-1),
                     omega_grid.astype(jnp.float32))
    return out9.T.reshape(_N, 3, 3)
